# Initial kernel scaffold; baseline (speedup 1.0000x reference)
import jax
import jax.numpy as jnp
from jax.experimental import pallas as pl


def kernel(edge_input_indices, edge_output_indices, edge_values, edge_type_embeddings, num_nodes, num_edge_types):
    raise NotImplementedError("write your pallas kernel here")


# speedup vs baseline: 1.6197x; 1.6197x over previous
import functools

import jax
import jax.numpy as jnp
from jax import lax
from jax.experimental import pallas as pl
from jax.experimental.pallas import tpu as pltpu
from jax.experimental.pallas import tpu_sc as plsc

N_NODES = 1024
N_TYPES = 16
N_EDGES = 131072
EMBED_DIM = 32
LANES = 16

R_SLAB = 6560
ROWS_PER_PASS = 32 * R_SLAB
N_PASSES = 5
W_PAD_ROWS = N_PASSES * ROWS_PER_PASS
CHUNK = 1024
N_CHUNKS = N_EDGES // CHUNK
GROUPS = CHUNK // LANES
C3 = 3 * CHUNK


@functools.cache
def _sc_build_w_fn():
    mesh = plsc.VectorSubcoreMesh(
        core_axis_name="c", subcore_axis_name="s")
    return functools.partial(
        pl.kernel,
        out_type=jax.ShapeDtypeStruct((W_PAD_ROWS * N_TYPES,), jnp.float32),
        mesh=mesh,
        compiler_params=pltpu.CompilerParams(needs_layout_passes=False),
        scratch_types=[
            pltpu.VMEM((R_SLAB * N_TYPES,), jnp.float32),
            pltpu.VMEM((2 * C3,), jnp.int32),
            pltpu.VMEM((2 * CHUNK,), jnp.float32),
            pltpu.SemaphoreType.DMA,
            pltpu.SemaphoreType.DMA,
        ],
    )(_sc_build_w_body)


def _sc_build_w_body(pk_hbm, pv_hbm, w_hbm, slab, stage, vstage, sem0, sem1):
    c = lax.axis_index("c")
    s = lax.axis_index("s")
    wid = c * 16 + s

    zeros16 = jnp.zeros((LANES,), jnp.float32)

    def make_process(base, b):
        toff = b * C3
        ioff = toff + CHUNK
        joff = toff + 2 * CHUNK
        voff = b * CHUNK

        def process():
            def group(u, _):
                e = u * LANES
                t = stage[pl.ds(toff + e, LANES)]
                i = stage[pl.ds(ioff + e, LANES)]
                j = stage[pl.ds(joff + e, LANES)]
                v = vstage[pl.ds(voff + e, LANES)]
                fwd = t < 8
                r = jnp.where(fwd, (i << 10) + j, (j << 10) + i)
                loc = r - base
                m = (loc >= 0) & (loc < R_SLAB)
                pos = (loc << 4) + t
                plsc.addupdate_scatter(slab, [pos], v, mask=m)
                return 0
            lax.fori_loop(0, GROUPS, group, 0, unroll=4)
        return process

    def start(k, b, sem):
        pltpu.async_copy(pk_hbm.at[pl.ds(k * C3, C3)],
                         stage.at[pl.ds(b * C3, C3)], sem)
        pltpu.async_copy(pv_hbm.at[pl.ds(k * CHUNK, CHUNK)],
                         vstage.at[pl.ds(b * CHUNK, CHUNK)], sem)

    def wait(k, b, sem):
        pltpu.make_async_copy(pk_hbm.at[pl.ds(k * C3, C3)],
                              stage.at[pl.ds(b * C3, C3)], sem).wait()
        pltpu.make_async_copy(pv_hbm.at[pl.ds(k * CHUNK, CHUNK)],
                              vstage.at[pl.ds(b * CHUNK, CHUNK)], sem).wait()

    for p in range(N_PASSES):
        base = p * ROWS_PER_PASS + wid * R_SLAB
        process0 = make_process(base, 0)
        process1 = make_process(base, 1)

        def zero_row(k, _):
            slab[pl.ds(k * LANES, LANES)] = zeros16
            return 0
        lax.fori_loop(0, R_SLAB, zero_row, 0, unroll=8)

        start(0, 0, sem0)

        def pair(g, _):
            k0 = g * 2
            start(k0 + 1, 1, sem1)
            wait(k0, 0, sem0)
            process0()

            @pl.when(g < N_CHUNKS // 2 - 1)
            def _():
                start(k0 + 2, 0, sem0)

            wait(k0 + 1, 1, sem1)
            process1()
            return 0

        lax.fori_loop(0, N_CHUNKS // 2, pair, 0)

        pltpu.sync_copy(slab, w_hbm.at[pl.ds(base * N_TYPES,
                                             R_SLAB * N_TYPES)])


_MM_BLK = 8192


def _mm_body(w_ref, e_ref, o_ref):
    o_ref[...] = jnp.dot(w_ref[...], e_ref[...],
                         preferred_element_type=jnp.float32)


def _tc_matmul(w, emb):
    return pl.pallas_call(
        _mm_body,
        grid=(N_NODES * N_NODES // _MM_BLK,),
        in_specs=[
            pl.BlockSpec((_MM_BLK, N_TYPES), lambda k: (k, 0)),
            pl.BlockSpec((N_TYPES, EMBED_DIM), lambda k: (0, 0)),
        ],
        out_specs=pl.BlockSpec((_MM_BLK, EMBED_DIM), lambda k: (k, 0)),
        out_shape=jax.ShapeDtypeStruct((N_NODES * N_NODES, EMBED_DIM),
                                       jnp.float32),
    )(w, emb)


def kernel(edge_input_indices, edge_output_indices, edge_values,
           edge_type_embeddings, num_nodes, num_edge_types):
    t = edge_input_indices[:, 0]
    i = edge_output_indices[:, 0]
    j = edge_output_indices[:, 1]
    pk = (jnp.stack([t, i, j], axis=0)
          .reshape(3, N_CHUNKS, CHUNK)
          .transpose(1, 0, 2)
          .reshape(-1))
    pv = edge_values
    w = _sc_build_w_fn()(pk, pv).reshape(W_PAD_ROWS, N_TYPES)
    out = _tc_matmul(w[:N_NODES * N_NODES], edge_type_embeddings)
    return out.reshape(N_NODES, N_NODES, EMBED_DIM)


# speedup vs baseline: 2.3485x; 1.4500x over previous
import functools

import jax
import jax.numpy as jnp
from jax import lax
from jax.experimental import pallas as pl
from jax.experimental.pallas import tpu as pltpu
from jax.experimental.pallas import tpu_sc as plsc

N_NODES = 1024
N_TYPES = 16
N_EDGES = 131072
EMBED_DIM = 32
LANES = 16

SEG = N_EDGES // 16
SEG_GROUPS = SEG // LANES
N_PASSES = 8
SLAB_WORDS = 1 << 20
STRIPE = SLAB_WORDS // 16
W_WORDS = 1 << 24
FIRE = 128
CAP = SEG + 2 * FIRE


@functools.cache
def _sc_build_w_fn():
    mesh = plsc.VectorSubcoreMesh(core_axis_name="c", subcore_axis_name="s")
    return functools.partial(
        pl.kernel,
        out_type=jax.ShapeDtypeStruct((W_WORDS,), jnp.float32),
        mesh=mesh,
        compiler_params=pltpu.CompilerParams(needs_layout_passes=False),
        scratch_types=[
            pltpu.VMEM_SHARED((SLAB_WORDS + LANES,), jnp.float32),
            pltpu.VMEM((SEG,), jnp.int32),
            pltpu.VMEM((SEG,), jnp.int32),
            pltpu.VMEM((SEG,), jnp.int32),
            pltpu.VMEM((SEG,), jnp.float32),
            pltpu.VMEM((SEG,), jnp.int32),
            pltpu.VMEM((CAP,), jnp.int32),
            pltpu.VMEM((CAP,), jnp.float32),
            pltpu.VMEM((FIRE,), jnp.int32),
        ],
    )(_sc_build_w_body)


def _sc_build_w_body(t_hbm, i_hbm, j_hbm, v_hbm, z_hbm, w_hbm,
                     slab, tb, ib, jb, vb, pg, pc, vc, idxb):
    c = lax.axis_index("c")
    s = lax.axis_index("s")
    iota16 = lax.iota(jnp.int32, LANES)

    seg = pl.multiple_of(s * SEG, 512)
    stripe_off = pl.multiple_of(s * STRIPE, 512)
    pltpu.sync_copy(t_hbm.at[pl.ds(seg, SEG)], tb)
    pltpu.sync_copy(i_hbm.at[pl.ds(seg, SEG)], ib)
    pltpu.sync_copy(j_hbm.at[pl.ds(seg, SEG)], jb)
    pltpu.sync_copy(v_hbm.at[pl.ds(seg, SEG)], vb)

    def pre(u, _):
        e = u * LANES
        t = tb[pl.ds(e, LANES)]
        i = ib[pl.ds(e, LANES)]
        j = jb[pl.ds(e, LANES)]
        r = jnp.where(t < 8, (i << 10) + j, (j << 10) + i)
        pg[pl.ds(e, LANES)] = (r << 4) + t
        return 0
    lax.fori_loop(0, SEG_GROUPS, pre, 0, unroll=4)

    pltpu.sync_copy(z_hbm, slab.at[pl.ds(stripe_off, STRIPE)])
    plsc.subcore_barrier()

    for p in range(N_PASSES):
        slab_id = p * 2 + c
        base = slab_id << 20

        def scan(u, cur):
            e = u * LANES
            q = pg[pl.ds(e, LANES)]
            v = vb[pl.ds(e, LANES)]
            m = (q >> 20) == slab_id
            plsc.store_compressed(pc.at[pl.ds(cur, LANES)], q, mask=m)
            plsc.store_compressed(vc.at[pl.ds(cur, LANES)], v, mask=m)
            return cur + jnp.sum(m.astype(jnp.int32))
        cnt = lax.fori_loop(0, SEG_GROUPS, scan, 0, unroll=4)

        padv = base + SLAB_WORDS + iota16
        for k in range(FIRE // LANES):
            pc[pl.ds(cnt + k * LANES, LANES)] = padv

        nf = (cnt + FIRE - 1) >> 7

        def fire(f, _):
            fb = f * FIRE
            for k in range(FIRE // LANES):
                q = pc[pl.ds(fb + k * LANES, LANES)]
                idxb[pl.ds(k * LANES, LANES)] = q - base
            pltpu.sync_copy(vc.at[pl.ds(fb, FIRE)], slab.at[idxb], add=True)
            return 0
        lax.fori_loop(0, nf, fire, 0)

        plsc.subcore_barrier()

        woff = pl.multiple_of(base + s * STRIPE, 512)
        pltpu.sync_copy(slab.at[pl.ds(stripe_off, STRIPE)],
                        w_hbm.at[pl.ds(woff, STRIPE)])
        if p < N_PASSES - 1:
            pltpu.sync_copy(z_hbm, slab.at[pl.ds(stripe_off, STRIPE)])
        plsc.subcore_barrier()


_MM_BLK = 8192


def _mm_body(w_ref, e_ref, o_ref):
    o_ref[...] = jnp.dot(w_ref[...], e_ref[...],
                         preferred_element_type=jnp.float32)


def _tc_matmul(w, emb):
    return pl.pallas_call(
        _mm_body,
        grid=(N_NODES * N_NODES // _MM_BLK,),
        in_specs=[
            pl.BlockSpec((_MM_BLK, N_TYPES), lambda k: (k, 0)),
            pl.BlockSpec((N_TYPES, EMBED_DIM), lambda k: (0, 0)),
        ],
        out_specs=pl.BlockSpec((_MM_BLK, EMBED_DIM), lambda k: (k, 0)),
        out_shape=jax.ShapeDtypeStruct((N_NODES * N_NODES, EMBED_DIM),
                                       jnp.float32),
    )(w, emb)


def kernel(edge_input_indices, edge_output_indices, edge_values,
           edge_type_embeddings, num_nodes, num_edge_types):
    t = edge_input_indices[:, 0]
    i = edge_output_indices[:, 0]
    j = edge_output_indices[:, 1]
    zeros = jnp.zeros((STRIPE,), jnp.float32)
    w = _sc_build_w_fn()(t, i, j, edge_values, zeros)
    out = _tc_matmul(w.reshape(N_NODES * N_NODES, N_TYPES),
                     edge_type_embeddings)
    return out.reshape(N_NODES, N_NODES, EMBED_DIM)


# speedup vs baseline: 2.8088x; 1.1960x over previous
import functools

import jax
import jax.numpy as jnp
from jax import lax
from jax.experimental import pallas as pl
from jax.experimental.pallas import tpu as pltpu
from jax.experimental.pallas import tpu_sc as plsc

N_NODES = 1024
N_TYPES = 16
N_EDGES = 131072
EMBED_DIM = 32
LANES = 16

SEG = N_EDGES // 16
SEG_GROUPS = SEG // LANES
N_PASSES = 8
SLAB_WORDS = 1 << 20
STRIPE = SLAB_WORDS // 16
W_WORDS = 1 << 24
FIRE = 128
CAP = SEG + 2 * FIRE


@functools.cache
def _sc_build_w_fn():
    mesh = plsc.VectorSubcoreMesh(core_axis_name="c", subcore_axis_name="s")
    return functools.partial(
        pl.kernel,
        out_type=jax.ShapeDtypeStruct((W_WORDS,), jnp.float32),
        mesh=mesh,
        compiler_params=pltpu.CompilerParams(needs_layout_passes=False),
        scratch_types=[
            pltpu.VMEM_SHARED((SLAB_WORDS + LANES,), jnp.float32),
            pltpu.VMEM((SEG,), jnp.int32),
            pltpu.VMEM((2 * SEG,), jnp.int32),
            pltpu.VMEM((SEG,), jnp.float32),
            pltpu.VMEM((SEG,), jnp.int32),
            pltpu.VMEM((CAP,), jnp.int32),
            pltpu.VMEM((CAP,), jnp.float32),
            pltpu.VMEM((FIRE,), jnp.int32),
        ],
    )(_sc_build_w_body)


def _sc_build_w_body(t_hbm, eo_hbm, v_hbm, z_hbm, w_hbm,
                     slab, tb, eb, vb, pg, pc, vc, idxb):
    c = lax.axis_index("c")
    s = lax.axis_index("s")
    iota16 = lax.iota(jnp.int32, LANES)

    seg = pl.multiple_of(s * SEG, 512)
    seg2 = pl.multiple_of(s * (2 * SEG), 512)
    stripe_off = pl.multiple_of(s * STRIPE, 512)
    pltpu.sync_copy(t_hbm.at[pl.ds(seg, SEG)], tb)
    pltpu.sync_copy(eo_hbm.at[pl.ds(seg2, 2 * SEG)], eb)
    pltpu.sync_copy(v_hbm.at[pl.ds(seg, SEG)], vb)

    def pre(u, _):
        e = u * LANES
        t = tb[pl.ds(e, LANES)]
        idx2 = u * (2 * LANES) + iota16 * 2
        i = plsc.load_gather(eb, [idx2])
        j = plsc.load_gather(eb, [idx2 + 1])
        r = jnp.where(t < 8, (i << 10) + j, (j << 10) + i)
        pg[pl.ds(e, LANES)] = (r << 4) + t
        return 0
    lax.fori_loop(0, SEG_GROUPS, pre, 0, unroll=4)

    pltpu.sync_copy(z_hbm, slab.at[pl.ds(stripe_off, STRIPE)])
    plsc.subcore_barrier()

    for p in range(N_PASSES):
        slab_id = p * 2 + c
        base = slab_id << 20

        def scan(u, cur):
            e = u * LANES
            q = pg[pl.ds(e, LANES)]
            v = vb[pl.ds(e, LANES)]
            m = (q >> 20) == slab_id
            plsc.store_compressed(pc.at[pl.ds(cur, LANES)], q, mask=m)
            plsc.store_compressed(vc.at[pl.ds(cur, LANES)], v, mask=m)
            return cur + jnp.sum(m.astype(jnp.int32))
        cnt = lax.fori_loop(0, SEG_GROUPS, scan, 0, unroll=4)

        padv = base + SLAB_WORDS + iota16
        for k in range(FIRE // LANES):
            pc[pl.ds(cnt + k * LANES, LANES)] = padv

        nf = (cnt + FIRE - 1) >> 7

        def fire(f, _):
            fb = f * FIRE
            for k in range(FIRE // LANES):
                q = pc[pl.ds(fb + k * LANES, LANES)]
                idxb[pl.ds(k * LANES, LANES)] = q - base
            pltpu.sync_copy(vc.at[pl.ds(fb, FIRE)], slab.at[idxb], add=True)
            return 0
        lax.fori_loop(0, nf, fire, 0)

        plsc.subcore_barrier()

        woff = pl.multiple_of(base + s * STRIPE, 512)
        pltpu.sync_copy(slab.at[pl.ds(stripe_off, STRIPE)],
                        w_hbm.at[pl.ds(woff, STRIPE)])
        if p < N_PASSES - 1:
            pltpu.sync_copy(z_hbm, slab.at[pl.ds(stripe_off, STRIPE)])
        plsc.subcore_barrier()


_MM_ROWS = W_WORDS // 128
_MM_BLK = 2048


def _mm_body(w_ref, e_ref, o_ref):
    o_ref[...] = jnp.dot(w_ref[...], e_ref[...],
                         preferred_element_type=jnp.float32)


def _tc_matmul(w128, big_emb):
    return pl.pallas_call(
        _mm_body,
        grid=(_MM_ROWS // _MM_BLK,),
        in_specs=[
            pl.BlockSpec((_MM_BLK, 128), lambda k: (k, 0)),
            pl.BlockSpec((128, 8 * EMBED_DIM), lambda k: (0, 0)),
        ],
        out_specs=pl.BlockSpec((_MM_BLK, 8 * EMBED_DIM), lambda k: (k, 0)),
        out_shape=jax.ShapeDtypeStruct((_MM_ROWS, 8 * EMBED_DIM),
                                       jnp.float32),
    )(w128, big_emb)


def kernel(edge_input_indices, edge_output_indices, edge_values,
           edge_type_embeddings, num_nodes, num_edge_types):
    t = edge_input_indices.reshape(-1)
    eo = edge_output_indices.reshape(-1)
    zeros = jnp.zeros((STRIPE,), jnp.float32)
    w = _sc_build_w_fn()(t, eo, edge_values, zeros)
    big_emb = jnp.kron(jnp.eye(8, dtype=jnp.float32), edge_type_embeddings)
    out = _tc_matmul(w.reshape(_MM_ROWS, 128), big_emb)
    return out.reshape(N_NODES, N_NODES, EMBED_DIM)


# speedup vs baseline: 2.8129x; 1.0015x over previous
import functools

import jax
import jax.numpy as jnp
from jax import lax
from jax.experimental import pallas as pl
from jax.experimental.pallas import tpu as pltpu
from jax.experimental.pallas import tpu_sc as plsc

N_NODES = 1024
N_TYPES = 16
N_EDGES = 131072
EMBED_DIM = 32
LANES = 16

SEG = N_EDGES // 16
SEG_GROUPS = SEG // LANES
N_PASSES = 8
SLAB_WORDS = 1 << 20
STRIPE = SLAB_WORDS // 16
W_WORDS = 1 << 24
FIRE = 128
CAP = SEG + 2 * FIRE


@functools.cache
def _sc_build_w_fn():
    mesh = plsc.VectorSubcoreMesh(core_axis_name="c", subcore_axis_name="s")
    return functools.partial(
        pl.kernel,
        out_type=jax.ShapeDtypeStruct((W_WORDS,), jnp.float32),
        mesh=mesh,
        compiler_params=pltpu.CompilerParams(needs_layout_passes=False),
        scratch_types=[
            pltpu.VMEM_SHARED((SLAB_WORDS + LANES,), jnp.float32),
            pltpu.VMEM((SEG,), jnp.int32),
            pltpu.VMEM((2 * SEG,), jnp.int32),
            pltpu.VMEM((SEG,), jnp.float32),
            pltpu.VMEM((SEG,), jnp.int32),
            pltpu.VMEM((CAP,), jnp.int32),
            pltpu.VMEM((CAP,), jnp.float32),
            pltpu.VMEM((FIRE,), jnp.int32),
        ],
    )(_sc_build_w_body)


def _sc_build_w_body(t_hbm, eo_hbm, v_hbm, z_hbm, w_hbm,
                     slab, tb, eb, vb, pg, pc, vc, idxb):
    c = lax.axis_index("c")
    s = lax.axis_index("s")
    iota16 = lax.iota(jnp.int32, LANES)

    seg = pl.multiple_of(s * SEG, 512)
    seg2 = pl.multiple_of(s * (2 * SEG), 512)
    stripe_off = pl.multiple_of(s * STRIPE, 512)
    pltpu.sync_copy(t_hbm.at[pl.ds(seg, SEG)], tb)
    pltpu.sync_copy(eo_hbm.at[pl.ds(seg2, 2 * SEG)], eb)
    pltpu.sync_copy(v_hbm.at[pl.ds(seg, SEG)], vb)

    def pre(u, _):
        e = u * LANES
        t = tb[pl.ds(e, LANES)]
        idx2 = u * (2 * LANES) + iota16 * 2
        i = plsc.load_gather(eb, [idx2])
        j = plsc.load_gather(eb, [idx2 + 1])
        r = jnp.where(t < 8, (i << 10) + j, (j << 10) + i)
        pg[pl.ds(e, LANES)] = (r << 4) + t
        return 0
    lax.fori_loop(0, SEG_GROUPS, pre, 0, unroll=4)

    pltpu.sync_copy(z_hbm, slab.at[pl.ds(stripe_off, STRIPE)])
    plsc.subcore_barrier()

    for p in range(N_PASSES):
        slab_id = p * 2 + c
        base = slab_id << 20

        def scan(u, cur):
            e = u * LANES
            q = pg[pl.ds(e, LANES)]
            v = vb[pl.ds(e, LANES)]
            m = (q >> 20) == slab_id
            plsc.store_compressed(pc.at[pl.ds(cur, LANES)], q, mask=m)
            plsc.store_compressed(vc.at[pl.ds(cur, LANES)], v, mask=m)
            return cur + jnp.sum(m.astype(jnp.int32))
        cnt = lax.fori_loop(0, SEG_GROUPS, scan, 0, unroll=4)

        padv = base + SLAB_WORDS + iota16
        for k in range(FIRE // LANES):
            pc[pl.ds(cnt + k * LANES, LANES)] = padv

        nf = (cnt + FIRE - 1) >> 7

        def fire(f, _):
            fb = f * FIRE
            for k in range(FIRE // LANES):
                q = pc[pl.ds(fb + k * LANES, LANES)]
                idxb[pl.ds(k * LANES, LANES)] = q - base
            pltpu.sync_copy(vc.at[pl.ds(fb, FIRE)], slab.at[idxb], add=True)
            return 0
        lax.fori_loop(0, nf, fire, 0)

        plsc.subcore_barrier()

        woff = pl.multiple_of(base + s * STRIPE, 512)
        pltpu.sync_copy(slab.at[pl.ds(stripe_off, STRIPE)],
                        w_hbm.at[pl.ds(woff, STRIPE)])
        if p < N_PASSES - 1:
            pltpu.sync_copy(z_hbm, slab.at[pl.ds(stripe_off, STRIPE)])
        plsc.subcore_barrier()


_MM_ROWS = W_WORDS // 128
_MM_BLK = 2048


def _mm_body(w_ref, e_ref, o_ref):
    w128 = w_ref[...].reshape(_MM_BLK, 128)
    o_ref[...] = jnp.dot(w128, e_ref[...], preferred_element_type=jnp.float32)


def _tc_matmul(w_flat, big_emb):
    return pl.pallas_call(
        _mm_body,
        grid=(_MM_ROWS // _MM_BLK,),
        in_specs=[
            pl.BlockSpec((_MM_BLK * 128,), lambda k: (k,)),
            pl.BlockSpec((128, 8 * EMBED_DIM), lambda k: (0, 0)),
        ],
        out_specs=pl.BlockSpec((_MM_BLK, 8 * EMBED_DIM), lambda k: (k, 0)),
        out_shape=jax.ShapeDtypeStruct((_MM_ROWS, 8 * EMBED_DIM),
                                       jnp.float32),
    )(w_flat, big_emb)


def kernel(edge_input_indices, edge_output_indices, edge_values,
           edge_type_embeddings, num_nodes, num_edge_types):
    t = edge_input_indices.reshape(-1)
    eo = edge_output_indices.reshape(-1)
    zeros = jnp.zeros((STRIPE,), jnp.float32)
    w = _sc_build_w_fn()(t, eo, edge_values, zeros)
    big_emb = jnp.kron(jnp.eye(8, dtype=jnp.float32), edge_type_embeddings)
    out = _tc_matmul(w, big_emb)
    return out.reshape(N_NODES, N_NODES, EMBED_DIM)


# speedup vs baseline: 6.5965x; 2.3451x over previous
import functools

import jax
import jax.numpy as jnp
from jax import lax
from jax.experimental import pallas as pl
from jax.experimental.pallas import tpu as pltpu
from jax.experimental.pallas import tpu_sc as plsc

N_NODES = 1024
N_TYPES = 16
N_EDGES = 131072
EMBED_DIM = 32
LANES = 16

SEG = N_EDGES // 16
SEG_GROUPS = SEG // LANES
N_PASSES = 16
SLAB_R = 1 << 15
SLAB_WORDS = N_TYPES * SLAB_R
STRIPE = SLAB_R
W_WORDS = 1 << 24
FIRE = 128
CAP = SEG + 2 * FIRE


@functools.cache
def _sc_build_w_fn():
    mesh = plsc.VectorSubcoreMesh(core_axis_name="c", subcore_axis_name="s")
    return functools.partial(
        pl.kernel,
        out_type=jax.ShapeDtypeStruct((W_WORDS,), jnp.float32),
        mesh=mesh,
        compiler_params=pltpu.CompilerParams(needs_layout_passes=False),
        scratch_types=[
            pltpu.VMEM_SHARED((SLAB_WORDS + LANES,), jnp.float32),
            pltpu.VMEM((SEG,), jnp.int32),
            pltpu.VMEM((SEG,), jnp.int32),
            pltpu.VMEM((SEG,), jnp.int32),
            pltpu.VMEM((SEG,), jnp.float32),
            pltpu.VMEM((SEG,), jnp.int32),
            pltpu.VMEM((SEG,), jnp.int32),
            pltpu.VMEM((CAP,), jnp.int32),
            pltpu.VMEM((CAP,), jnp.float32),
            pltpu.VMEM((FIRE,), jnp.int32),
        ],
    )(_sc_build_w_body)


def _sc_build_w_body(t_hbm, i_hbm, j_hbm, v_hbm, z_hbm, w_hbm,
                     slab, tb, ib, jb, vb, sel, loc, pc, vc, idxb):
    c = lax.axis_index("c")
    s = lax.axis_index("s")
    iota16 = lax.iota(jnp.int32, LANES)

    seg = pl.multiple_of(s * SEG, 512)
    stripe_off = pl.multiple_of(s * STRIPE, 512)
    pltpu.sync_copy(t_hbm.at[pl.ds(seg, SEG)], tb)
    pltpu.sync_copy(i_hbm.at[pl.ds(seg, SEG)], ib)
    pltpu.sync_copy(j_hbm.at[pl.ds(seg, SEG)], jb)
    pltpu.sync_copy(v_hbm.at[pl.ds(seg, SEG)], vb)

    def pre(u, _):
        e = u * LANES
        t = tb[pl.ds(e, LANES)]
        i = ib[pl.ds(e, LANES)]
        j = jb[pl.ds(e, LANES)]
        r = jnp.where(t < 8, (i << 10) + j, (j << 10) + i)
        sel[pl.ds(e, LANES)] = r >> 15
        loc[pl.ds(e, LANES)] = (t << 15) | (r & 0x7FFF)
        return 0
    lax.fori_loop(0, SEG_GROUPS, pre, 0, unroll=4)

    pltpu.sync_copy(z_hbm, slab.at[pl.ds(stripe_off, STRIPE)])
    plsc.subcore_barrier()

    for p in range(N_PASSES):
        slab_id = p * 2 + c

        def scan(u, cur):
            e = u * LANES
            q = loc[pl.ds(e, LANES)]
            v = vb[pl.ds(e, LANES)]
            m = sel[pl.ds(e, LANES)] == slab_id
            plsc.store_compressed(pc.at[pl.ds(cur, LANES)], q, mask=m)
            plsc.store_compressed(vc.at[pl.ds(cur, LANES)], v, mask=m)
            return cur + jnp.sum(m.astype(jnp.int32))
        cnt = lax.fori_loop(0, SEG_GROUPS, scan, 0, unroll=4)

        padv = SLAB_WORDS + iota16
        for k in range(FIRE // LANES):
            pc[pl.ds(cnt + k * LANES, LANES)] = padv

        nf = (cnt + FIRE - 1) >> 7

        def fire(f, _):
            fb = f * FIRE
            for k in range(FIRE // LANES):
                idxb[pl.ds(k * LANES, LANES)] = pc[pl.ds(fb + k * LANES,
                                                         LANES)]
            pltpu.sync_copy(vc.at[pl.ds(fb, FIRE)], slab.at[idxb], add=True)
            return 0
        lax.fori_loop(0, nf, fire, 0)

        plsc.subcore_barrier()

        woff = pl.multiple_of(s * (1 << 20) + slab_id * STRIPE, 512)
        pltpu.sync_copy(slab.at[pl.ds(stripe_off, STRIPE)],
                        w_hbm.at[pl.ds(woff, STRIPE)])
        if p < N_PASSES - 1:
            pltpu.sync_copy(z_hbm, slab.at[pl.ds(stripe_off, STRIPE)])
        plsc.subcore_barrier()


_MM_B0 = 16
_MM_RSPAN = _MM_B0 * N_NODES


def _mm_body(*refs):
    w_refs = refs[:N_TYPES]
    e_ref = refs[N_TYPES]
    o_ref = refs[N_TYPES + 1]
    vs = [w_refs[tt][...].reshape(1, _MM_RSPAN) for tt in range(N_TYPES)]
    v = jnp.concatenate(vs, axis=0)
    acc = jnp.dot(e_ref[...], v, preferred_element_type=jnp.float32)
    for g in range(_MM_B0):
        o_ref[g, :, :] = lax.slice(acc, (0, g * N_NODES),
                                   (EMBED_DIM, (g + 1) * N_NODES))


def _tc_matmul(w_flat, emb_t):
    n_steps = (N_NODES * N_NODES) // _MM_RSPAN
    in_specs = [
        pl.BlockSpec((_MM_RSPAN,), functools.partial(
            lambda tt, k: (tt * n_steps + k,), tt))
        for tt in range(N_TYPES)
    ]
    in_specs.append(pl.BlockSpec((EMBED_DIM, N_TYPES), lambda k: (0, 0)))
    return pl.pallas_call(
        _mm_body,
        grid=(n_steps,),
        in_specs=in_specs,
        out_specs=pl.BlockSpec((_MM_B0, EMBED_DIM, N_NODES),
                               lambda k: (k, 0, 0)),
        out_shape=jax.ShapeDtypeStruct((N_NODES, EMBED_DIM, N_NODES),
                                       jnp.float32),
    )(*([w_flat] * N_TYPES), emb_t)


def kernel(edge_input_indices, edge_output_indices, edge_values,
           edge_type_embeddings, num_nodes, num_edge_types):
    t = edge_input_indices[:, 0]
    i = edge_output_indices[:, 0]
    j = edge_output_indices[:, 1]
    zeros = jnp.zeros((STRIPE,), jnp.float32)
    w = _sc_build_w_fn()(t, i, j, edge_values, zeros)
    out_t = _tc_matmul(w, edge_type_embeddings.T)
    return out_t.transpose(0, 2, 1)


# speedup vs baseline: 7.0843x; 1.0740x over previous
import functools

import jax
import jax.numpy as jnp
from jax import lax
from jax.experimental import pallas as pl
from jax.experimental.pallas import tpu as pltpu
from jax.experimental.pallas import tpu_sc as plsc

N_NODES = 1024
N_TYPES = 16
N_EDGES = 131072
EMBED_DIM = 32
LANES = 16

SEG = N_EDGES // 16
SEG_GROUPS = SEG // LANES
N_PASSES = 16
SLAB_R = 1 << 15
SLAB_WORDS = N_TYPES * SLAB_R
STRIPE = SLAB_R
W_WORDS = 1 << 24
FIRE = 128
CAP = SEG + 2 * FIRE


@functools.cache
def _sc_build_w_fn():
    mesh = plsc.VectorSubcoreMesh(core_axis_name="c", subcore_axis_name="s")
    return functools.partial(
        pl.kernel,
        out_type=jax.ShapeDtypeStruct((W_WORDS,), jnp.float32),
        mesh=mesh,
        compiler_params=pltpu.CompilerParams(needs_layout_passes=False),
        scratch_types=[
            pltpu.VMEM_SHARED((SLAB_WORDS + LANES,), jnp.float32),
            pltpu.VMEM((SEG,), jnp.int32),
            pltpu.VMEM((SEG,), jnp.int32),
            pltpu.VMEM((SEG,), jnp.int32),
            pltpu.VMEM((SEG,), jnp.float32),
            pltpu.VMEM((SEG,), jnp.int32),
            pltpu.VMEM((SEG,), jnp.int32),
            pltpu.VMEM((CAP,), jnp.int32),
            pltpu.VMEM((CAP,), jnp.float32),
            pltpu.VMEM((FIRE,), jnp.int32),
        ],
    )(_sc_build_w_body)


def _sc_build_w_body(t_hbm, i_hbm, j_hbm, v_hbm, z_hbm, w_hbm,
                     slab, tb, ib, jb, vb, sel, loc, pc, vc, idxb):
    c = lax.axis_index("c")
    s = lax.axis_index("s")
    iota16 = lax.iota(jnp.int32, LANES)

    seg = pl.multiple_of(s * SEG, 512)
    stripe_off = pl.multiple_of(s * STRIPE, 512)
    pltpu.sync_copy(t_hbm.at[pl.ds(seg, SEG)], tb)
    pltpu.sync_copy(i_hbm.at[pl.ds(seg, SEG)], ib)
    pltpu.sync_copy(j_hbm.at[pl.ds(seg, SEG)], jb)
    pltpu.sync_copy(v_hbm.at[pl.ds(seg, SEG)], vb)

    def pre(u, _):
        e = u * LANES
        t = tb[pl.ds(e, LANES)]
        i = ib[pl.ds(e, LANES)]
        j = jb[pl.ds(e, LANES)]
        r = jnp.where(t < 8, (i << 10) + j, (j << 10) + i)
        sel[pl.ds(e, LANES)] = r >> 15
        loc[pl.ds(e, LANES)] = (t << 15) | (r & 0x7FFF)
        return 0
    lax.fori_loop(0, SEG_GROUPS, pre, 0, unroll=4)

    pltpu.sync_copy(z_hbm, slab.at[pl.ds(stripe_off, STRIPE)])
    plsc.subcore_barrier()

    for p in range(N_PASSES):
        slab_id = p * 2 + c

        def scan(u, cur):
            e = u * LANES
            q = loc[pl.ds(e, LANES)]
            v = vb[pl.ds(e, LANES)]
            m = sel[pl.ds(e, LANES)] == slab_id
            plsc.store_compressed(pc.at[pl.ds(cur, LANES)], q, mask=m)
            plsc.store_compressed(vc.at[pl.ds(cur, LANES)], v, mask=m)
            return cur + jnp.sum(m.astype(jnp.int32))
        cnt = lax.fori_loop(0, SEG_GROUPS, scan, 0, unroll=4)

        padv = SLAB_WORDS + iota16
        for k in range(FIRE // LANES):
            pc[pl.ds(cnt + k * LANES, LANES)] = padv

        nf = (cnt + FIRE - 1) >> 7

        def fire(f, _):
            fb = f * FIRE
            for k in range(FIRE // LANES):
                idxb[pl.ds(k * LANES, LANES)] = pc[pl.ds(fb + k * LANES,
                                                         LANES)]
            pltpu.sync_copy(vc.at[pl.ds(fb, FIRE)], slab.at[idxb], add=True)
            return 0
        lax.fori_loop(0, nf, fire, 0)

        plsc.subcore_barrier()

        woff = pl.multiple_of(s * (1 << 20) + slab_id * STRIPE, 512)
        pltpu.sync_copy(slab.at[pl.ds(stripe_off, STRIPE)],
                        w_hbm.at[pl.ds(woff, STRIPE)])
        if p < N_PASSES - 1:
            pltpu.sync_copy(z_hbm, slab.at[pl.ds(stripe_off, STRIPE)])
        plsc.subcore_barrier()


_MM_B0 = 32
_MM_RSPAN = _MM_B0 * N_NODES


def _mm_body(*refs):
    w_refs = refs[:N_TYPES]
    e_ref = refs[N_TYPES]
    o_ref = refs[N_TYPES + 1]
    vs = [w_refs[tt][...].reshape(1, _MM_RSPAN) for tt in range(N_TYPES)]
    v = jnp.concatenate(vs, axis=0)
    acc = jnp.dot(e_ref[...], v, preferred_element_type=jnp.float32)
    for g in range(_MM_B0):
        o_ref[g, :, :] = lax.slice(acc, (0, g * N_NODES),
                                   (EMBED_DIM, (g + 1) * N_NODES))


def _tc_matmul(w_flat, emb_t):
    n_steps = (N_NODES * N_NODES) // _MM_RSPAN
    in_specs = [
        pl.BlockSpec((_MM_RSPAN,), functools.partial(
            lambda tt, k: (tt * n_steps + k,), tt))
        for tt in range(N_TYPES)
    ]
    in_specs.append(pl.BlockSpec((EMBED_DIM, N_TYPES), lambda k: (0, 0)))
    return pl.pallas_call(
        _mm_body,
        grid=(n_steps,),
        in_specs=in_specs,
        out_specs=pl.BlockSpec((_MM_B0, EMBED_DIM, N_NODES),
                               lambda k: (k, 0, 0)),
        out_shape=jax.ShapeDtypeStruct((N_NODES, EMBED_DIM, N_NODES),
                                       jnp.float32),
    )(*([w_flat] * N_TYPES), emb_t)


def kernel(edge_input_indices, edge_output_indices, edge_values,
           edge_type_embeddings, num_nodes, num_edge_types):
    t = edge_input_indices[:, 0]
    i = edge_output_indices[:, 0]
    j = edge_output_indices[:, 1]
    zeros = jnp.zeros((STRIPE,), jnp.float32)
    w = _sc_build_w_fn()(t, i, j, edge_values, zeros)
    out_t = _tc_matmul(w, edge_type_embeddings.T)
    return out_t.transpose(0, 2, 1)


# speedup vs baseline: 9.1972x; 1.2983x over previous
import functools

import jax
import jax.numpy as jnp
from jax import lax
from jax.experimental import pallas as pl
from jax.experimental.pallas import tpu as pltpu
from jax.experimental.pallas import tpu_sc as plsc

N_NODES = 1024
N_TYPES = 16
N_EDGES = 131072
EMBED_DIM = 32
LANES = 16

SEG = N_EDGES // 16
SEG_GROUPS = SEG // LANES
N_PASSES = 16
SLAB_R = 1 << 15
SLAB_WORDS = N_TYPES * SLAB_R
STRIPE = SLAB_R
W_WORDS = 1 << 24
FIRE = 128
CAP = SEG + 2 * FIRE


@functools.cache
def _sc_build_w_fn():
    mesh = plsc.VectorSubcoreMesh(core_axis_name="c", subcore_axis_name="s")
    return functools.partial(
        pl.kernel,
        out_type=jax.ShapeDtypeStruct((W_WORDS,), jnp.float32),
        mesh=mesh,
        compiler_params=pltpu.CompilerParams(needs_layout_passes=False),
        scratch_types=[
            pltpu.VMEM_SHARED((SLAB_WORDS + LANES,), jnp.float32),
            pltpu.VMEM((SEG,), jnp.int32),
            pltpu.VMEM((SEG,), jnp.int32),
            pltpu.VMEM((SEG,), jnp.int32),
            pltpu.VMEM((SEG,), jnp.float32),
            pltpu.VMEM((SEG,), jnp.int32),
            pltpu.VMEM((SEG,), jnp.int32),
            pltpu.VMEM((CAP,), jnp.int32),
            pltpu.VMEM((CAP,), jnp.float32),
            pltpu.VMEM((FIRE,), jnp.int32),
            pltpu.SemaphoreType.DMA,
            pltpu.SemaphoreType.DMA,
        ],
    )(_sc_build_w_body)


def _sc_build_w_body(t_hbm, i_hbm, j_hbm, v_hbm, z_hbm, w_hbm,
                     slab, tb, ib, jb, vb, sel, loc, pc, vc, idxb,
                     fsem, zsem):
    c = lax.axis_index("c")
    s = lax.axis_index("s")
    iota16 = lax.iota(jnp.int32, LANES)

    seg = pl.multiple_of(s * SEG, 512)
    stripe_off = pl.multiple_of(s * STRIPE, 512)
    pltpu.sync_copy(t_hbm.at[pl.ds(seg, SEG)], tb)
    pltpu.sync_copy(i_hbm.at[pl.ds(seg, SEG)], ib)
    pltpu.sync_copy(j_hbm.at[pl.ds(seg, SEG)], jb)
    pltpu.sync_copy(v_hbm.at[pl.ds(seg, SEG)], vb)

    def pre(u, _):
        e = u * LANES
        t = tb[pl.ds(e, LANES)]
        i = ib[pl.ds(e, LANES)]
        j = jb[pl.ds(e, LANES)]
        r = jnp.where(t < 8, (i << 10) + j, (j << 10) + i)
        sel[pl.ds(e, LANES)] = r >> 15
        loc[pl.ds(e, LANES)] = (t << 15) | (r & 0x7FFF)
        return 0
    lax.fori_loop(0, SEG_GROUPS, pre, 0, unroll=4)

    def woff_of(p):
        return pl.multiple_of(s * (1 << 20) + (p * 2 + c) * STRIPE, 512)

    def scan_span(slab_id, lo, hi, cur0):
        def scan(u, cur):
            e = u * LANES
            q = loc[pl.ds(e, LANES)]
            v = vb[pl.ds(e, LANES)]
            m = sel[pl.ds(e, LANES)] == slab_id
            plsc.store_compressed(pc.at[pl.ds(cur, LANES)], q, mask=m)
            plsc.store_compressed(vc.at[pl.ds(cur, LANES)], v, mask=m)
            return cur + jnp.sum(m.astype(jnp.int32))
        return lax.fori_loop(lo, hi, scan, cur0, unroll=4)

    pltpu.sync_copy(z_hbm, slab.at[pl.ds(stripe_off, STRIPE)])
    plsc.subcore_barrier()

    for p in range(N_PASSES):
        slab_id = p * 2 + c

        cnt = scan_span(slab_id, 0, SEG_GROUPS // 2, 0)
        if p >= 1:
            pltpu.make_async_copy(
                slab.at[pl.ds(stripe_off, STRIPE)],
                w_hbm.at[pl.ds(woff_of(p - 1), STRIPE)], fsem).wait()
            pltpu.async_copy(z_hbm, slab.at[pl.ds(stripe_off, STRIPE)], zsem)
        cnt = scan_span(slab_id, SEG_GROUPS // 2, SEG_GROUPS, cnt)
        if p >= 1:
            pltpu.make_async_copy(
                z_hbm, slab.at[pl.ds(stripe_off, STRIPE)], zsem).wait()
        plsc.subcore_barrier()

        padv = SLAB_WORDS + iota16
        for k in range(FIRE // LANES):
            pc[pl.ds(cnt + k * LANES, LANES)] = padv

        nf = (cnt + FIRE - 1) >> 7

        def fire(f, _):
            fb = f * FIRE
            for k in range(FIRE // LANES):
                idxb[pl.ds(k * LANES, LANES)] = pc[pl.ds(fb + k * LANES,
                                                         LANES)]
            pltpu.sync_copy(vc.at[pl.ds(fb, FIRE)], slab.at[idxb], add=True)
            return 0
        lax.fori_loop(0, nf, fire, 0)

        plsc.subcore_barrier()
        pltpu.async_copy(slab.at[pl.ds(stripe_off, STRIPE)],
                         w_hbm.at[pl.ds(woff_of(p), STRIPE)], fsem)

    pltpu.make_async_copy(
        slab.at[pl.ds(stripe_off, STRIPE)],
        w_hbm.at[pl.ds(woff_of(N_PASSES - 1), STRIPE)], fsem).wait()


_MM_B0 = 32
_MM_RSPAN = _MM_B0 * N_NODES


def _mm_body(*refs):
    w_refs = refs[:N_TYPES]
    e_ref = refs[N_TYPES]
    o_ref = refs[N_TYPES + 1]
    vs = [w_refs[tt][...].reshape(1, _MM_RSPAN) for tt in range(N_TYPES)]
    v = jnp.concatenate(vs, axis=0)
    acc = jnp.dot(e_ref[...], v, preferred_element_type=jnp.float32)
    for g in range(_MM_B0):
        o_ref[g, :, :] = lax.slice(acc, (0, g * N_NODES),
                                   (EMBED_DIM, (g + 1) * N_NODES))


def _tc_matmul(w_flat, emb_t):
    n_steps = (N_NODES * N_NODES) // _MM_RSPAN
    in_specs = [
        pl.BlockSpec((_MM_RSPAN,), functools.partial(
            lambda tt, k: (tt * n_steps + k,), tt))
        for tt in range(N_TYPES)
    ]
    in_specs.append(pl.BlockSpec((EMBED_DIM, N_TYPES), lambda k: (0, 0)))
    return pl.pallas_call(
        _mm_body,
        grid=(n_steps,),
        in_specs=in_specs,
        out_specs=pl.BlockSpec((_MM_B0, EMBED_DIM, N_NODES),
                               lambda k: (k, 0, 0)),
        out_shape=jax.ShapeDtypeStruct((N_NODES, EMBED_DIM, N_NODES),
                                       jnp.float32),
    )(*([w_flat] * N_TYPES), emb_t)


def kernel(edge_input_indices, edge_output_indices, edge_values,
           edge_type_embeddings, num_nodes, num_edge_types):
    t = edge_input_indices[:, 0]
    i = edge_output_indices[:, 0]
    j = edge_output_indices[:, 1]
    zeros = jnp.zeros((STRIPE,), jnp.float32)
    w = _sc_build_w_fn()(t, i, j, edge_values, zeros)
    out_t = _tc_matmul(w, edge_type_embeddings.T)
    return out_t.transpose(0, 2, 1)


# speedup vs baseline: 9.8878x; 1.0751x over previous
import functools

import jax
import jax.numpy as jnp
from jax import lax
from jax.experimental import pallas as pl
from jax.experimental.pallas import tpu as pltpu
from jax.experimental.pallas import tpu_sc as plsc

N_NODES = 1024
N_TYPES = 16
N_EDGES = 131072
EMBED_DIM = 32
LANES = 16

SEG = N_EDGES // 16
SEG_GROUPS = SEG // LANES
N_PASSES = 8
HALF_R = 1 << 19
SLAB_R = 1 << 15
SLAB_WORDS = N_TYPES * SLAB_R
STRIPE = SLAB_R
W_WORDS = 1 << 24
FIRE = 128
CAP = SEG + 2 * FIRE


@functools.cache
def _sc_build_w_fn(half):
    mesh = plsc.VectorSubcoreMesh(core_axis_name="c", subcore_axis_name="s")
    return functools.partial(
        pl.kernel,
        out_type=jax.ShapeDtypeStruct((W_WORDS // 2,), jnp.float32),
        mesh=mesh,
        compiler_params=pltpu.CompilerParams(needs_layout_passes=False),
        scratch_types=[
            pltpu.VMEM_SHARED((SLAB_WORDS + LANES,), jnp.float32),
            pltpu.VMEM((SEG,), jnp.int32),
            pltpu.VMEM((SEG,), jnp.int32),
            pltpu.VMEM((SEG,), jnp.int32),
            pltpu.VMEM((SEG,), jnp.float32),
            pltpu.VMEM((SEG,), jnp.int32),
            pltpu.VMEM((SEG,), jnp.int32),
            pltpu.VMEM((CAP,), jnp.int32),
            pltpu.VMEM((CAP,), jnp.float32),
            pltpu.VMEM((FIRE,), jnp.int32),
            pltpu.SemaphoreType.DMA,
            pltpu.SemaphoreType.DMA,
        ],
    )(functools.partial(_sc_build_w_body, half))


def _sc_build_w_body(half, t_hbm, i_hbm, j_hbm, v_hbm, z_hbm, w_hbm,
                     slab, tb, ib, jb, vb, sel, loc, pc, vc, idxb,
                     fsem, zsem):
    c = lax.axis_index("c")
    s = lax.axis_index("s")
    iota16 = lax.iota(jnp.int32, LANES)

    seg = pl.multiple_of(s * SEG, 512)
    stripe_off = pl.multiple_of(s * STRIPE, 512)
    pltpu.sync_copy(t_hbm.at[pl.ds(seg, SEG)], tb)
    pltpu.sync_copy(i_hbm.at[pl.ds(seg, SEG)], ib)
    pltpu.sync_copy(j_hbm.at[pl.ds(seg, SEG)], jb)
    pltpu.sync_copy(v_hbm.at[pl.ds(seg, SEG)], vb)

    def pre(u, _):
        e = u * LANES
        t = tb[pl.ds(e, LANES)]
        i = ib[pl.ds(e, LANES)]
        j = jb[pl.ds(e, LANES)]
        r = jnp.where(t < 8, (i << 10) + j, (j << 10) + i)
        sel[pl.ds(e, LANES)] = r >> 15
        loc[pl.ds(e, LANES)] = (t << 15) | (r & 0x7FFF)
        return 0
    lax.fori_loop(0, SEG_GROUPS, pre, 0, unroll=4)

    def woff_of(p):
        return pl.multiple_of(s * (1 << 19) + (p * 2 + c) * STRIPE, 512)

    def scan_span(slab_id, lo, hi, cur0):
        def scan(u, cur):
            e = u * LANES
            q = loc[pl.ds(e, LANES)]
            v = vb[pl.ds(e, LANES)]
            m = sel[pl.ds(e, LANES)] == slab_id
            plsc.store_compressed(pc.at[pl.ds(cur, LANES)], q, mask=m)
            plsc.store_compressed(vc.at[pl.ds(cur, LANES)], v, mask=m)
            return cur + jnp.sum(m.astype(jnp.int32))
        return lax.fori_loop(lo, hi, scan, cur0, unroll=4)

    pltpu.sync_copy(z_hbm, slab.at[pl.ds(stripe_off, STRIPE)])
    plsc.subcore_barrier()

    for p in range(N_PASSES):
        slab_id = (half * N_PASSES + p) * 2 + c

        cnt = scan_span(slab_id, 0, SEG_GROUPS // 2, 0)
        if p >= 1:
            pltpu.make_async_copy(
                slab.at[pl.ds(stripe_off, STRIPE)],
                w_hbm.at[pl.ds(woff_of(p - 1), STRIPE)], fsem).wait()
            pltpu.async_copy(z_hbm, slab.at[pl.ds(stripe_off, STRIPE)], zsem)
        cnt = scan_span(slab_id, SEG_GROUPS // 2, SEG_GROUPS, cnt)
        if p >= 1:
            pltpu.make_async_copy(
                z_hbm, slab.at[pl.ds(stripe_off, STRIPE)], zsem).wait()
        plsc.subcore_barrier()

        padv = SLAB_WORDS + iota16
        for k in range(FIRE // LANES):
            pc[pl.ds(cnt + k * LANES, LANES)] = padv

        nf = (cnt + FIRE - 1) >> 7

        def fire(f, _):
            fb = f * FIRE
            for k in range(FIRE // LANES):
                idxb[pl.ds(k * LANES, LANES)] = pc[pl.ds(fb + k * LANES,
                                                         LANES)]
            pltpu.sync_copy(vc.at[pl.ds(fb, FIRE)], slab.at[idxb], add=True)
            return 0
        lax.fori_loop(0, nf, fire, 0)

        plsc.subcore_barrier()
        pltpu.async_copy(slab.at[pl.ds(stripe_off, STRIPE)],
                         w_hbm.at[pl.ds(woff_of(p), STRIPE)], fsem)

    pltpu.make_async_copy(
        slab.at[pl.ds(stripe_off, STRIPE)],
        w_hbm.at[pl.ds(woff_of(N_PASSES - 1), STRIPE)], fsem).wait()


_MM_B0 = 32
_MM_RSPAN = _MM_B0 * N_NODES


def _mm_body(*refs):
    w_refs = refs[:N_TYPES]
    e_ref = refs[N_TYPES]
    o_ref = refs[-1]
    vs = [w_refs[tt][...].reshape(1, _MM_RSPAN) for tt in range(N_TYPES)]
    v = jnp.concatenate(vs, axis=0)
    acc = jnp.dot(e_ref[...], v, preferred_element_type=jnp.float32)
    for g in range(_MM_B0):
        o_ref[g, :, :] = lax.slice(acc, (0, g * N_NODES),
                                   (EMBED_DIM, (g + 1) * N_NODES))


def _tc_matmul(w_half, emb_t, half, prev=None):
    n_steps = (N_NODES // 2 * N_NODES) // _MM_RSPAN
    in_specs = [
        pl.BlockSpec((_MM_RSPAN,), functools.partial(
            lambda tt, k: (tt * n_steps + k,), tt))
        for tt in range(N_TYPES)
    ]
    in_specs.append(pl.BlockSpec((EMBED_DIM, N_TYPES), lambda k: (0, 0)))
    args = [w_half] * N_TYPES + [emb_t]
    kwargs = {}
    if prev is not None:
        in_specs.append(pl.BlockSpec(memory_space=pltpu.MemorySpace.HBM))
        args.append(prev)
        kwargs["input_output_aliases"] = {N_TYPES + 1: 0}

    def body(*refs):
        _mm_body(*refs[:N_TYPES + 1], refs[-1])

    return pl.pallas_call(
        body,
        grid=(n_steps,),
        in_specs=in_specs,
        out_specs=pl.BlockSpec((_MM_B0, EMBED_DIM, N_NODES),
                               lambda k: (half * n_steps + k, 0, 0)),
        out_shape=jax.ShapeDtypeStruct((N_NODES, EMBED_DIM, N_NODES),
                                       jnp.float32),
        **kwargs,
    )(*args)


def kernel(edge_input_indices, edge_output_indices, edge_values,
           edge_type_embeddings, num_nodes, num_edge_types):
    t = edge_input_indices[:, 0]
    i = edge_output_indices[:, 0]
    j = edge_output_indices[:, 1]
    zeros = jnp.zeros((STRIPE,), jnp.float32)
    emb_t = edge_type_embeddings.T
    w1 = _sc_build_w_fn(0)(t, i, j, edge_values, zeros)
    w2 = _sc_build_w_fn(1)(t, i, j, edge_values, zeros)
    out1 = _tc_matmul(w1, emb_t, 0)
    out_t = _tc_matmul(w2, emb_t, 1, prev=out1)
    return out_t.transpose(0, 2, 1)


# speedup vs baseline: 10.5975x; 1.0718x over previous
import functools

import jax
import jax.numpy as jnp
from jax import lax
from jax.experimental import pallas as pl
from jax.experimental.pallas import tpu as pltpu
from jax.experimental.pallas import tpu_sc as plsc

N_NODES = 1024
N_TYPES = 16
N_EDGES = 131072
EMBED_DIM = 32
LANES = 16

SEG = N_EDGES // 16
SEG_GROUPS = SEG // LANES
N_PASSES = 8
HALF_R = 1 << 19
SLAB_R = 1 << 15
SLAB_WORDS = N_TYPES * SLAB_R
STRIPE = SLAB_R
W_WORDS = 1 << 24
FIRE = 128
CAP = SEG + 2 * FIRE


@functools.cache
def _sc_build_w_fn(half):
    mesh = plsc.VectorSubcoreMesh(core_axis_name="c", subcore_axis_name="s")
    return functools.partial(
        pl.kernel,
        out_type=jax.ShapeDtypeStruct((W_WORDS // 2,), jnp.float32),
        mesh=mesh,
        compiler_params=pltpu.CompilerParams(needs_layout_passes=False),
        scratch_types=[
            pltpu.VMEM_SHARED((SLAB_WORDS + LANES,), jnp.float32),
            pltpu.VMEM((SEG,), jnp.int32),
            pltpu.VMEM((SEG,), jnp.int32),
            pltpu.VMEM((SEG,), jnp.int32),
            pltpu.VMEM((SEG,), jnp.float32),
            pltpu.VMEM((CAP,), jnp.int32),
            pltpu.VMEM((CAP,), jnp.int32),
            pltpu.VMEM((CAP,), jnp.float32),
            pltpu.VMEM((CAP,), jnp.int32),
            pltpu.VMEM((CAP,), jnp.float32),
            pltpu.VMEM((FIRE,), jnp.int32),
            pltpu.SemaphoreType.DMA,
            pltpu.SemaphoreType.DMA,
        ],
    )(functools.partial(_sc_build_w_body, half))


def _sc_build_w_body(half, t_hbm, i_hbm, j_hbm, v_hbm, z_hbm, w_hbm,
                     slab, tb, ib, jb, vb, sel, loc, vh, pc, vc, idxb,
                     fsem, zsem):
    c = lax.axis_index("c")
    s = lax.axis_index("s")
    iota16 = lax.iota(jnp.int32, LANES)

    seg = pl.multiple_of(s * SEG, 512)
    stripe_off = pl.multiple_of(s * STRIPE, 512)
    pltpu.sync_copy(t_hbm.at[pl.ds(seg, SEG)], tb)
    pltpu.sync_copy(i_hbm.at[pl.ds(seg, SEG)], ib)
    pltpu.sync_copy(j_hbm.at[pl.ds(seg, SEG)], jb)
    pltpu.sync_copy(v_hbm.at[pl.ds(seg, SEG)], vb)

    def pre(u, cur):
        e = u * LANES
        t = tb[pl.ds(e, LANES)]
        i = ib[pl.ds(e, LANES)]
        j = jb[pl.ds(e, LANES)]
        v = vb[pl.ds(e, LANES)]
        r = jnp.where(t < 8, (i << 10) + j, (j << 10) + i)
        m = (r >> 19) == half
        plsc.store_compressed(sel.at[pl.ds(cur, LANES)], r >> 15, mask=m)
        plsc.store_compressed(loc.at[pl.ds(cur, LANES)],
                              (t << 15) | (r & 0x7FFF), mask=m)
        plsc.store_compressed(vh.at[pl.ds(cur, LANES)], v, mask=m)
        return cur + jnp.sum(m.astype(jnp.int32))
    nh = lax.fori_loop(0, SEG_GROUPS, pre, 0, unroll=4)
    sent = jnp.full((LANES,), 255, jnp.int32)
    sel[pl.ds(nh, LANES)] = sent
    sel[pl.ds(nh + LANES, LANES)] = sent
    nh_groups = (nh + LANES - 1) >> 4

    def woff_of(p):
        return pl.multiple_of(s * (1 << 19) + (p * 2 + c) * STRIPE, 512)

    def scan_span(slab_id, lo, hi, cur0):
        def scan(u, cur):
            e = u * LANES
            q = loc[pl.ds(e, LANES)]
            v = vh[pl.ds(e, LANES)]
            m = sel[pl.ds(e, LANES)] == slab_id
            plsc.store_compressed(pc.at[pl.ds(cur, LANES)], q, mask=m)
            plsc.store_compressed(vc.at[pl.ds(cur, LANES)], v, mask=m)
            return cur + jnp.sum(m.astype(jnp.int32))
        return lax.fori_loop(lo, hi, scan, cur0)

    pltpu.sync_copy(z_hbm, slab.at[pl.ds(stripe_off, STRIPE)])
    plsc.subcore_barrier()

    for p in range(N_PASSES):
        slab_id = (half * N_PASSES + p) * 2 + c

        cnt = scan_span(slab_id, 0, nh_groups >> 1, 0)
        if p >= 1:
            pltpu.make_async_copy(
                slab.at[pl.ds(stripe_off, STRIPE)],
                w_hbm.at[pl.ds(woff_of(p - 1), STRIPE)], fsem).wait()
            pltpu.async_copy(z_hbm, slab.at[pl.ds(stripe_off, STRIPE)], zsem)
        cnt = scan_span(slab_id, nh_groups >> 1, nh_groups, cnt)
        if p >= 1:
            pltpu.make_async_copy(
                z_hbm, slab.at[pl.ds(stripe_off, STRIPE)], zsem).wait()
        plsc.subcore_barrier()

        padv = SLAB_WORDS + iota16
        for k in range(FIRE // LANES):
            pc[pl.ds(cnt + k * LANES, LANES)] = padv

        nf = (cnt + FIRE - 1) >> 7

        def fire(f, _):
            fb = f * FIRE
            for k in range(FIRE // LANES):
                idxb[pl.ds(k * LANES, LANES)] = pc[pl.ds(fb + k * LANES,
                                                         LANES)]
            pltpu.sync_copy(vc.at[pl.ds(fb, FIRE)], slab.at[idxb], add=True)
            return 0
        lax.fori_loop(0, nf, fire, 0)

        plsc.subcore_barrier()
        pltpu.async_copy(slab.at[pl.ds(stripe_off, STRIPE)],
                         w_hbm.at[pl.ds(woff_of(p), STRIPE)], fsem)

    pltpu.make_async_copy(
        slab.at[pl.ds(stripe_off, STRIPE)],
        w_hbm.at[pl.ds(woff_of(N_PASSES - 1), STRIPE)], fsem).wait()


_MM_B0 = 32
_MM_RSPAN = _MM_B0 * N_NODES


def _mm_body(*refs):
    w_refs = refs[:N_TYPES]
    e_ref = refs[N_TYPES]
    o_ref = refs[-1]
    vs = [w_refs[tt][...].reshape(1, _MM_RSPAN) for tt in range(N_TYPES)]
    v = jnp.concatenate(vs, axis=0)
    acc = jnp.dot(e_ref[...], v, preferred_element_type=jnp.float32)
    for g in range(_MM_B0):
        o_ref[g, :, :] = lax.slice(acc, (0, g * N_NODES),
                                   (EMBED_DIM, (g + 1) * N_NODES))


def _tc_matmul(w_half, emb_t, half, prev=None):
    n_steps = (N_NODES // 2 * N_NODES) // _MM_RSPAN
    in_specs = [
        pl.BlockSpec((_MM_RSPAN,), functools.partial(
            lambda tt, k: (tt * n_steps + k,), tt))
        for tt in range(N_TYPES)
    ]
    in_specs.append(pl.BlockSpec((EMBED_DIM, N_TYPES), lambda k: (0, 0)))
    args = [w_half] * N_TYPES + [emb_t]
    kwargs = {}
    if prev is not None:
        in_specs.append(pl.BlockSpec(memory_space=pltpu.MemorySpace.HBM))
        args.append(prev)
        kwargs["input_output_aliases"] = {N_TYPES + 1: 0}

    def body(*refs):
        _mm_body(*refs[:N_TYPES + 1], refs[-1])

    return pl.pallas_call(
        body,
        grid=(n_steps,),
        in_specs=in_specs,
        out_specs=pl.BlockSpec((_MM_B0, EMBED_DIM, N_NODES),
                               lambda k: (half * n_steps + k, 0, 0)),
        out_shape=jax.ShapeDtypeStruct((N_NODES, EMBED_DIM, N_NODES),
                                       jnp.float32),
        **kwargs,
    )(*args)


def kernel(edge_input_indices, edge_output_indices, edge_values,
           edge_type_embeddings, num_nodes, num_edge_types):
    t = edge_input_indices[:, 0]
    i = edge_output_indices[:, 0]
    j = edge_output_indices[:, 1]
    zeros = jnp.zeros((STRIPE,), jnp.float32)
    emb_t = edge_type_embeddings.T
    w1 = _sc_build_w_fn(0)(t, i, j, edge_values, zeros)
    w2 = _sc_build_w_fn(1)(t, i, j, edge_values, zeros)
    out1 = _tc_matmul(w1, emb_t, 0)
    out_t = _tc_matmul(w2, emb_t, 1, prev=out1)
    return out_t.transpose(0, 2, 1)
